# baseline (speedup 1.0000x reference)
import functools
import jax
import jax.numpy as jnp
from jax import lax
from jax.experimental import pallas as pl
from jax.experimental.pallas import tpu as pltpu
from jax.experimental.pallas import tpu_sc as plsc

_VOC = 1000000
_EMBED = 64
_WINDOW = 200
_BATCH = 1024
_SCALE = 8.0
_WT, _BT, _WS, _BL = _WINDOW // 8, _BATCH // 128, 8, 128

_VBLK = 256
_PITCH = 65
_NBLK = 999936 // _VBLK
_TAIL0 = 999936


def _k1_body(tt_hbm, tail_hbm, tp_hbm, tb0, tb1, rb0, rb1, tailv,
             gs0, gs1, os0, os1):
    tbufs, robufs = [tb0, tb1], [rb0, rb1]
    gsems, osems = [gs0, gs1], [os0, os1]
    wid = lax.axis_index("s") * 2 + lax.axis_index("c")
    lo = (_NBLK * wid) // 32
    hi = (_NBLK * (wid + 1)) // 32
    n = hi - lo
    lane = lax.iota(jnp.int32, 16)

    def gin_start(b, j):
        for s in range(8):
            pltpu.make_async_copy(
                tt_hbm.at[pl.ds(8 * s, 8), pl.ds(b * _VBLK, _VBLK)],
                tbufs[j].at[s], gsems[j]).start()

    def gin_wait(j):
        for s in range(8):
            pltpu.make_async_copy(
                tt_hbm.at[pl.ds(0, 8), pl.ds(0, _VBLK)],
                tbufs[j].at[s], gsems[j]).wait()

    def out_start(b, j):
        pltpu.make_async_copy(
            robufs[j], tp_hbm.at[pl.ds(b * _VBLK, _VBLK)], osems[j]).start()

    def out_wait(j):
        pltpu.make_async_copy(
            robufs[j], tp_hbm.at[pl.ds(0, _VBLK)], osems[j]).wait()

    @pl.when(n > 0)
    def _():
        gin_start(lo, 0)

    @pl.when(n > 1)
    def _():
        gin_start(lo + 1, 1)

    def step(i, j, b):
        gin_wait(j)

        @pl.when(i >= 2)
        def _():
            out_wait(j)

        def grp(g, carry):
            rows = 16 * g + lane
            for e in range(_EMBED):
                s, t = e // 8, e % 8
                vals = tbufs[j][s, t, pl.ds(16 * g, 16)]
                plsc.store_scatter(
                    robufs[j], [rows, jnp.full((16,), e, jnp.int32)], vals)
            return carry

        lax.fori_loop(0, 16, grp, 0)
        out_start(b, j)

        @pl.when(i + 2 < n)
        def _():
            gin_start(b + 2, j)

    def outer(i2, carry):
        for j in range(2):
            i = 2 * i2 + j

            @pl.when(i < n)
            def _():
                step(i, j, lo + i)
        return carry

    lax.fori_loop(0, (n + 1) // 2, outer, 0)

    @pl.when(n > 0)
    def _():
        out_wait(0)

    @pl.when(n > 1)
    def _():
        out_wait(1)

    @pl.when(wid == 31)
    def _():
        pltpu.sync_copy(tail_hbm, tailv)
        for vl in range(64):
            for q in range(4):
                rb0[vl, pl.ds(16 * q, 16)] = tailv[vl, pl.ds(16 * q, 16)]
        pltpu.sync_copy(rb0.at[pl.ds(0, 64)], tp_hbm.at[pl.ds(_TAIL0, 64)])


_k1 = functools.partial(
    pl.kernel,
    mesh=plsc.VectorSubcoreMesh(core_axis_name="c", subcore_axis_name="s"),
    out_type=jax.ShapeDtypeStruct((_VOC, _PITCH), jnp.float32),
    scratch_types=[
        pltpu.VMEM((8, 8, _VBLK), jnp.float32),
        pltpu.VMEM((8, 8, _VBLK), jnp.float32),
        pltpu.VMEM((_VBLK, _PITCH), jnp.float32),
        pltpu.VMEM((_VBLK, _PITCH), jnp.float32),
        pltpu.VMEM((64, _EMBED), jnp.float32),
        pltpu.SemaphoreType.DMA,
        pltpu.SemaphoreType.DMA,
        pltpu.SemaphoreType.DMA,
        pltpu.SemaphoreType.DMA,
    ],
    compiler_params=pltpu.CompilerParams(
        use_tc_tiling_on_sc=True, needs_layout_passes=False),
)(_k1_body)


def _k2_body(tp_hbm, idx_hbm, pos_hbm, q5_hbm,
             idxv, posv, pb, gb0, gb1, sb0, sb1,
             gs0, gs1, os0, os1):
    gbufs, sbufs = [gb0, gb1], [sb0, sb1]
    gsems, osems = [gs0, gs1], [os0, os1]
    wid = lax.axis_index("s") * 2 + lax.axis_index("c")

    @pl.when(wid < _WT)
    def _():
        pltpu.sync_copy(idx_hbm.at[wid], idxv)
        pltpu.sync_copy(pos_hbm, posv)
        lane = lax.iota(jnp.int32, 16)

        def g_start(u, j):
            ws, bt = u // 8, u % 8
            pltpu.make_async_copy(
                tp_hbm.at[idxv.at[bt, ws]], gbufs[j], gsems[j]).start()

        def g_wait(j):
            pltpu.make_async_copy(
                tp_hbm.at[idxv.at[0, 0]], gbufs[j], gsems[j]).wait()

        def o_start(u, j):
            ws, bt = u // 8, u % 8
            pltpu.make_async_copy(
                sbufs[j], q5_hbm.at[8 * wid + ws, :, bt], osems[j]).start()

        def o_wait(j):
            pltpu.make_async_copy(
                sbufs[j], q5_hbm.at[0, :, 0], osems[j]).wait()

        g_start(0, 0)
        g_start(1, 1)

        def unit(u, j):
            ws, bt = u // 8, u % 8
            w = 8 * wid + ws
            g_wait(j)

            @pl.when(u >= 2)
            def _():
                o_wait(j)

            @pl.when(bt == 0)
            def _():
                def bld(e, carry2):
                    pb[e, pl.ds(0, 16)] = plsc.load_gather(
                        posv, [jnp.full((16,), w, jnp.int32),
                               jnp.full((16,), e, jnp.int32)])
                    return carry2
                lax.fori_loop(0, _EMBED, bld, 0)

            def mgrp(m, carry2):
                rows = 16 * m + lane
                for e in range(_EMBED):
                    vals = plsc.load_gather(
                        gbufs[j], [rows, jnp.full((16,), e, jnp.int32)])
                    res = vals * _SCALE + pb[e, pl.ds(0, 16)]
                    sbufs[j][e // 8, e % 8, pl.ds(16 * m, 16)] = res
                return carry2

            lax.fori_loop(0, 8, mgrp, 0)
            o_start(u, j)

            @pl.when(u + 2 < 64)
            def _():
                g_start(u + 2, j)

        def upair(i2, carry):
            for j in range(2):
                unit(2 * i2 + j, j)
            return carry

        lax.fori_loop(0, 32, upair, 0)
        o_wait(0)
        o_wait(1)


_k2 = functools.partial(
    pl.kernel,
    mesh=plsc.VectorSubcoreMesh(core_axis_name="c", subcore_axis_name="s"),
    out_type=jax.ShapeDtypeStruct((_WINDOW, 8, _BT, 8, 128), jnp.float32),
    scratch_types=[
        pltpu.VMEM((_WS, _BT, 128), jnp.int32),
        pltpu.VMEM((_WINDOW, _EMBED), jnp.float32),
        pltpu.VMEM((_EMBED, 16), jnp.float32),
        pltpu.VMEM((128, _PITCH), jnp.float32),
        pltpu.VMEM((128, _PITCH), jnp.float32),
        pltpu.VMEM((8, 8, 128), jnp.float32),
        pltpu.VMEM((8, 8, 128), jnp.float32),
        pltpu.SemaphoreType.DMA,
        pltpu.SemaphoreType.DMA,
        pltpu.SemaphoreType.DMA,
        pltpu.SemaphoreType.DMA,
    ],
    compiler_params=pltpu.CompilerParams(
        use_tc_tiling_on_sc=False, needs_layout_passes=False),
)(_k2_body)


def kernel(x, table, pos_enc):
    x4 = jnp.transpose(
        jnp.reshape(jnp.transpose(x.astype(jnp.int32)), (_WT, _WS, _BT, _BL)),
        (0, 2, 1, 3))
    tt = jnp.transpose(table)
    tail = lax.slice(table, (_TAIL0, 0), (_VOC, _EMBED))
    tp = _k1(tt, tail)
    q5 = _k2(tp, x4, pos_enc)
    out = jnp.transpose(
        jnp.reshape(jnp.transpose(q5, (0, 1, 3, 2, 4)),
                    (_WINDOW, _EMBED, _BATCH)),
        (2, 0, 1))
    return out


# speedup vs baseline: 1.8028x; 1.8028x over previous
import functools
import jax
import jax.numpy as jnp
from jax import lax
from jax.experimental import pallas as pl
from jax.experimental.pallas import tpu as pltpu
from jax.experimental.pallas import tpu_sc as plsc

_VOC = 1000000
_EMBED = 64
_WINDOW = 200
_BATCH = 1024
_SCALE = 8.0
_WT, _BT, _WS, _BL = _WINDOW // 8, _BATCH // 128, 8, 128

_VBLK = 256
_NBLK = 999936 // _VBLK
_TAIL0 = 999936


def _k1_body(tt_hbm, tail_hbm, tp_hbm, tb0, tb1, rb0, rb1, tailv,
             gs0, gs1, os0, os1):
    tbufs, robufs = [tb0, tb1], [rb0, rb1]
    gsems, osems = [gs0, gs1], [os0, os1]
    wid = lax.axis_index("s") * 2 + lax.axis_index("c")
    lo = (_NBLK * wid) // 32
    hi = (_NBLK * (wid + 1)) // 32
    n = hi - lo
    lane = lax.iota(jnp.int32, 16)

    def gin_start(b, j):
        for s in range(8):
            pltpu.make_async_copy(
                tt_hbm.at[pl.ds(8 * s, 8), pl.ds(b * _VBLK, _VBLK)],
                tbufs[j].at[pl.ds(8 * s, 8)], gsems[j]).start()

    def gin_wait(j):
        for s in range(8):
            pltpu.make_async_copy(
                tt_hbm.at[pl.ds(0, 8), pl.ds(0, _VBLK)],
                tbufs[j].at[pl.ds(0, 8)], gsems[j]).wait()

    def out_start(b, j):
        pltpu.make_async_copy(
            robufs[j], tp_hbm.at[pl.ds(b * _VBLK, _VBLK)], osems[j]).start()

    def out_wait(j):
        pltpu.make_async_copy(
            robufs[j], tp_hbm.at[pl.ds(0, _VBLK)], osems[j]).wait()

    @pl.when(n > 0)
    def _():
        gin_start(lo, 0)

    @pl.when(n > 1)
    def _():
        gin_start(lo + 1, 1)

    def step(i, j, b):
        gin_wait(j)

        @pl.when(i >= 2)
        def _():
            out_wait(j)

        def grp(g, carry):
            rows = 16 * g + lane
            for e0 in range(_EMBED):
                cols = (e0 + lane) & 63
                vals = plsc.load_gather(tbufs[j], [cols, rows])
                plsc.store_scatter(robufs[j], [rows, cols], vals)
            return carry

        lax.fori_loop(0, 16, grp, 0)
        out_start(b, j)

        @pl.when(i + 2 < n)
        def _():
            gin_start(b + 2, j)

    def outer(i2, carry):
        for j in range(2):
            i = 2 * i2 + j

            @pl.when(i < n)
            def _():
                step(i, j, lo + i)
        return carry

    lax.fori_loop(0, (n + 1) // 2, outer, 0)

    @pl.when(n > 0)
    def _():
        out_wait(0)

    @pl.when(n > 1)
    def _():
        out_wait(1)

    @pl.when(wid == 31)
    def _():
        pltpu.sync_copy(tail_hbm, tailv)
        for vl in range(64):
            for q in range(4):
                rb0[vl, pl.ds(16 * q, 16)] = tailv[vl, pl.ds(16 * q, 16)]
        pltpu.sync_copy(rb0.at[pl.ds(0, 64)], tp_hbm.at[pl.ds(_TAIL0, 64)])


_k1 = functools.partial(
    pl.kernel,
    mesh=plsc.VectorSubcoreMesh(core_axis_name="c", subcore_axis_name="s"),
    out_type=jax.ShapeDtypeStruct((_VOC, _EMBED), jnp.float32),
    scratch_types=[
        pltpu.VMEM((_EMBED, _VBLK), jnp.float32),
        pltpu.VMEM((_EMBED, _VBLK), jnp.float32),
        pltpu.VMEM((_VBLK, _EMBED), jnp.float32),
        pltpu.VMEM((_VBLK, _EMBED), jnp.float32),
        pltpu.VMEM((64, _EMBED), jnp.float32),
        pltpu.SemaphoreType.DMA,
        pltpu.SemaphoreType.DMA,
        pltpu.SemaphoreType.DMA,
        pltpu.SemaphoreType.DMA,
    ],
    compiler_params=pltpu.CompilerParams(
        use_tc_tiling_on_sc=True, needs_layout_passes=False),
)(_k1_body)


def _k2_body(tp_hbm, idx_hbm, pos_hbm, q5_hbm,
             idxv, posv, pb, gb0, gb1, sb0, sb1,
             gs0, gs1, os0, os1):
    gbufs, sbufs = [gb0, gb1], [sb0, sb1]
    gsems, osems = [gs0, gs1], [os0, os1]
    wid = lax.axis_index("s") * 2 + lax.axis_index("c")

    @pl.when(wid < _WT)
    def _():
        pltpu.sync_copy(idx_hbm.at[wid], idxv)
        pltpu.sync_copy(pos_hbm, posv)
        lane = lax.iota(jnp.int32, 16)

        def g_start(u, j):
            ws, bt = u // 8, u % 8
            pltpu.make_async_copy(
                tp_hbm.at[idxv.at[bt, ws]], gbufs[j], gsems[j]).start()

        def g_wait(j):
            pltpu.make_async_copy(
                tp_hbm.at[idxv.at[0, 0]], gbufs[j], gsems[j]).wait()

        def o_start(u, j):
            ws, bt = u // 8, u % 8
            pltpu.make_async_copy(
                sbufs[j], q5_hbm.at[8 * wid + ws, :, bt], osems[j]).start()

        def o_wait(j):
            pltpu.make_async_copy(
                sbufs[j], q5_hbm.at[0, :, 0], osems[j]).wait()

        g_start(0, 0)
        g_start(1, 1)

        def unit(u, j):
            ws, bt = u // 8, u % 8
            w = 8 * wid + ws
            g_wait(j)

            @pl.when(u >= 2)
            def _():
                o_wait(j)

            @pl.when(bt == 0)
            def _():
                def bld(e0, carry2):
                    pb[e0, pl.ds(0, 16)] = plsc.load_gather(
                        posv, [jnp.full((16,), w, jnp.int32),
                               (e0 + lane) & 63])
                    return carry2
                lax.fori_loop(0, _EMBED, bld, 0)

            def mgrp(m, carry2):
                rows = 16 * m + lane
                for e0 in range(_EMBED):
                    cols = (e0 + lane) & 63
                    vals = plsc.load_gather(gbufs[j], [rows, cols])
                    res = vals * _SCALE + pb[e0, pl.ds(0, 16)]
                    plsc.store_scatter(
                        sbufs[j],
                        [lax.shift_right_logical(cols, 3), cols & 7, rows],
                        res)
                return carry2

            lax.fori_loop(0, 8, mgrp, 0)
            o_start(u, j)

            @pl.when(u + 2 < 64)
            def _():
                g_start(u + 2, j)

        def upair(i2, carry):
            for j in range(2):
                unit(2 * i2 + j, j)
            return carry

        lax.fori_loop(0, 32, upair, 0)
        o_wait(0)
        o_wait(1)


_k2 = functools.partial(
    pl.kernel,
    mesh=plsc.VectorSubcoreMesh(core_axis_name="c", subcore_axis_name="s"),
    out_type=jax.ShapeDtypeStruct((_WINDOW, 8, _BT, 8, 128), jnp.float32),
    scratch_types=[
        pltpu.VMEM((_WS, _BT, 128), jnp.int32),
        pltpu.VMEM((_WINDOW, _EMBED), jnp.float32),
        pltpu.VMEM((_EMBED, 16), jnp.float32),
        pltpu.VMEM((128, _EMBED), jnp.float32),
        pltpu.VMEM((128, _EMBED), jnp.float32),
        pltpu.VMEM((8, 8, 128), jnp.float32),
        pltpu.VMEM((8, 8, 128), jnp.float32),
        pltpu.SemaphoreType.DMA,
        pltpu.SemaphoreType.DMA,
        pltpu.SemaphoreType.DMA,
        pltpu.SemaphoreType.DMA,
    ],
    compiler_params=pltpu.CompilerParams(
        use_tc_tiling_on_sc=False, needs_layout_passes=False),
)(_k2_body)


def kernel(x, table, pos_enc):
    x4 = jnp.transpose(
        jnp.reshape(jnp.transpose(x.astype(jnp.int32)), (_WT, _WS, _BT, _BL)),
        (0, 2, 1, 3))
    tt = jnp.transpose(table)
    tail = lax.slice(table, (_TAIL0, 0), (_VOC, _EMBED))
    tp = _k1(tt, tail)
    q5 = _k2(tp, x4, pos_enc)
    out = jnp.transpose(
        jnp.reshape(jnp.transpose(q5, (0, 1, 3, 2, 4)),
                    (_WINDOW, _EMBED, _BATCH)),
        (2, 0, 1))
    return out


# speedup vs baseline: 3.0050x; 1.6668x over previous
import functools
import jax
import jax.numpy as jnp
from jax import lax
from jax.experimental import pallas as pl
from jax.experimental.pallas import tpu as pltpu
from jax.experimental.pallas import tpu_sc as plsc

_EMBED = 64
_WINDOW = 200
_BATCH = 1024
_NW = 32
_SPW = _BATCH // _NW
_CHUNK = _WINDOW
_NBUF = 4
_OUTER = _SPW // _NBUF
_SCALE = 8.0


def _body(idx_hbm, table_hbm, pos_hbm, out_hbm,
          idx_v, pos_v, b0, b1, b2, b3,
          g0, g1, g2, g3, o0, o1, o2, o3):
    bufs = [b0, b1, b2, b3]
    gsems = [g0, g1, g2, g3]
    osems = [o0, o1, o2, o3]
    wid = lax.axis_index("s") * 2 + lax.axis_index("c")
    seq0 = wid * _SPW

    pltpu.sync_copy(idx_hbm.at[pl.ds(seq0, _SPW)], idx_v)
    pltpu.sync_copy(pos_hbm, pos_v)

    def gather_start(c, j):
        pltpu.make_async_copy(
            table_hbm.at[idx_v.at[c]], bufs[j], gsems[j]).start()

    def gather_wait(j):
        pltpu.make_async_copy(
            table_hbm.at[idx_v.at[0]], bufs[j], gsems[j]).wait()

    def scatter_start(c, j):
        pltpu.make_async_copy(
            bufs[j], out_hbm.at[seq0 + c], osems[j]).start()

    def scatter_wait(j):
        pltpu.make_async_copy(
            bufs[j], out_hbm.at[0], osems[j]).wait()

    gather_start(0, 0)
    gather_start(1, 1)

    def compute(j):
        buf = bufs[j]

        def row(r, carry):
            for q in range(_EMBED // 16):
                sl = (r, pl.ds(q * 16, 16))
                buf[sl] = buf[sl] * _SCALE + pos_v[sl]
            return carry

        lax.fori_loop(0, _CHUNK, row, 0)

    def outer(i, carry):
        for j in range(_NBUF):
            c = i * _NBUF + j
            gather_wait(j)
            compute(j)
            scatter_start(c, j)
            jn = (j + 2) % _NBUF
            if j < 2:
                @pl.when(i >= 1)
                def _():
                    scatter_wait(jn)
                gather_start(c + 2, jn)
            else:
                @pl.when(i < _OUTER - 1)
                def _():
                    scatter_wait(jn)
                    gather_start(c + 2, jn)
        return carry

    lax.fori_loop(0, _OUTER, outer, 0)
    for j in range(_NBUF):
        scatter_wait(j)


_sc_call = functools.partial(
    pl.kernel,
    mesh=plsc.VectorSubcoreMesh(core_axis_name="c", subcore_axis_name="s"),
    out_type=jax.ShapeDtypeStruct((_BATCH, _WINDOW, _EMBED), jnp.float32),
    scratch_types=[
        pltpu.VMEM((_SPW, _WINDOW), jnp.int32),
        pltpu.VMEM((_WINDOW, _EMBED), jnp.float32),
        pltpu.VMEM((_CHUNK, _EMBED), jnp.float32),
        pltpu.VMEM((_CHUNK, _EMBED), jnp.float32),
        pltpu.VMEM((_CHUNK, _EMBED), jnp.float32),
        pltpu.VMEM((_CHUNK, _EMBED), jnp.float32),
        pltpu.SemaphoreType.DMA,
        pltpu.SemaphoreType.DMA,
        pltpu.SemaphoreType.DMA,
        pltpu.SemaphoreType.DMA,
        pltpu.SemaphoreType.DMA,
        pltpu.SemaphoreType.DMA,
        pltpu.SemaphoreType.DMA,
        pltpu.SemaphoreType.DMA,
    ],
    compiler_params=pltpu.CompilerParams(use_tc_tiling_on_sc=False),
)(_body)


def kernel(x, table, pos_enc):
    return _sc_call(x.astype(jnp.int32), table, pos_enc)


# speedup vs baseline: 3.2447x; 1.0798x over previous
import functools
import jax
import jax.numpy as jnp
from jax import lax
from jax.experimental import pallas as pl
from jax.experimental.pallas import tpu as pltpu
from jax.experimental.pallas import tpu_sc as plsc

_VOC = 1000000
_EMBED = 64
_WINDOW = 200
_BATCH = 1024
_SCALE = 8.0
_WT, _BT, _WS, _BL = _WINDOW // 8, _BATCH // 128, 8, 128

_VBLK = 256
_NBLK = 999936 // _VBLK
_TAIL0 = 999936


def _k1_body(tt_hbm, tail_hbm, tp_hbm, tb0, tb1, rb0, rb1, tailv,
             gs0, gs1, os0, os1):
    tbufs, robufs = [tb0, tb1], [rb0, rb1]
    gsems, osems = [gs0, gs1], [os0, os1]
    wid = lax.axis_index("s") * 2 + lax.axis_index("c")
    lo = (_NBLK * wid) // 32
    hi = (_NBLK * (wid + 1)) // 32
    n = hi - lo
    lane = lax.iota(jnp.int32, 16)

    def gin_start(b, j):
        for s in range(8):
            pltpu.make_async_copy(
                tt_hbm.at[pl.ds(8 * s, 8), pl.ds(b * _VBLK, _VBLK)],
                tbufs[j].at[pl.ds(8 * s, 8)], gsems[j]).start()

    def gin_wait(j):
        for s in range(8):
            pltpu.make_async_copy(
                tt_hbm.at[pl.ds(0, 8), pl.ds(0, _VBLK)],
                tbufs[j].at[pl.ds(0, 8)], gsems[j]).wait()

    def out_start(b, j):
        pltpu.make_async_copy(
            robufs[j], tp_hbm.at[pl.ds(b * _VBLK, _VBLK)], osems[j]).start()

    def out_wait(j):
        pltpu.make_async_copy(
            robufs[j], tp_hbm.at[pl.ds(0, _VBLK)], osems[j]).wait()

    @pl.when(n > 0)
    def _():
        gin_start(lo, 0)

    @pl.when(n > 1)
    def _():
        gin_start(lo + 1, 1)

    def step(i, j, b):
        gin_wait(j)

        @pl.when(i >= 2)
        def _():
            out_wait(j)

        @functools.partial(plsc.parallel_loop, 0, 16, unroll=2)
        def _(g):
            rows = 16 * g + lane
            for e0 in range(_EMBED):
                cols = (e0 + lane) & 63
                vals = plsc.load_gather(tbufs[j], [cols, rows])
                plsc.store_scatter(robufs[j], [rows, cols], vals)

        out_start(b, j)

        @pl.when(i + 2 < n)
        def _():
            gin_start(b + 2, j)

    def outer(i2, carry):
        for j in range(2):
            i = 2 * i2 + j

            @pl.when(i < n)
            def _():
                step(i, j, lo + i)
        return carry

    lax.fori_loop(0, (n + 1) // 2, outer, 0)

    @pl.when(n > 0)
    def _():
        out_wait(0)

    @pl.when(n > 1)
    def _():
        out_wait(1)

    @pl.when(wid == 31)
    def _():
        pltpu.sync_copy(tail_hbm, tailv)
        for vl in range(64):
            for q in range(4):
                rb0[vl, pl.ds(16 * q, 16)] = tailv[vl, pl.ds(16 * q, 16)]
        pltpu.sync_copy(rb0.at[pl.ds(0, 64)], tp_hbm.at[pl.ds(_TAIL0, 64)])


_k1 = functools.partial(
    pl.kernel,
    mesh=plsc.VectorSubcoreMesh(core_axis_name="c", subcore_axis_name="s"),
    out_type=jax.ShapeDtypeStruct((_VOC, _EMBED), jnp.float32),
    scratch_types=[
        pltpu.VMEM((_EMBED, _VBLK), jnp.float32),
        pltpu.VMEM((_EMBED, _VBLK), jnp.float32),
        pltpu.VMEM((_VBLK, _EMBED), jnp.float32),
        pltpu.VMEM((_VBLK, _EMBED), jnp.float32),
        pltpu.VMEM((64, _EMBED), jnp.float32),
        pltpu.SemaphoreType.DMA,
        pltpu.SemaphoreType.DMA,
        pltpu.SemaphoreType.DMA,
        pltpu.SemaphoreType.DMA,
    ],
    compiler_params=pltpu.CompilerParams(
        use_tc_tiling_on_sc=True, needs_layout_passes=False),
)(_k1_body)


def _k2_body(tp_hbm, idx_hbm, pos_hbm, q5_hbm,
             idxv, posv, pb, gb0, gb1, sb0, sb1,
             gs0, gs1, os0, os1):
    gbufs, sbufs = [gb0, gb1], [sb0, sb1]
    gsems, osems = [gs0, gs1], [os0, os1]
    wid = lax.axis_index("s") * 2 + lax.axis_index("c")

    @pl.when(wid < _WT)
    def _():
        pltpu.sync_copy(idx_hbm.at[wid], idxv)
        pltpu.sync_copy(pos_hbm, posv)
        lane = lax.iota(jnp.int32, 16)

        def g_start(u, j):
            ws, bt = u // 8, u % 8
            pltpu.make_async_copy(
                tp_hbm.at[idxv.at[bt, ws]], gbufs[j], gsems[j]).start()

        def g_wait(j):
            pltpu.make_async_copy(
                tp_hbm.at[idxv.at[0, 0]], gbufs[j], gsems[j]).wait()

        def o_start(u, j):
            ws, bt = u // 8, u % 8
            pltpu.make_async_copy(
                sbufs[j], q5_hbm.at[8 * wid + ws, :, bt], osems[j]).start()

        def o_wait(j):
            pltpu.make_async_copy(
                sbufs[j], q5_hbm.at[0, :, 0], osems[j]).wait()

        g_start(0, 0)
        g_start(1, 1)

        def unit(u, j):
            ws, bt = u // 8, u % 8
            w = 8 * wid + ws
            g_wait(j)

            @pl.when(u >= 2)
            def _():
                o_wait(j)

            @pl.when(bt == 0)
            def _():
                def bld(e0, carry2):
                    pb[e0, pl.ds(0, 16)] = plsc.load_gather(
                        posv, [jnp.full((16,), w, jnp.int32),
                               (e0 + lane) & 63])
                    return carry2
                lax.fori_loop(0, _EMBED, bld, 0)

            @functools.partial(plsc.parallel_loop, 0, 8, unroll=2)
            def _(m):
                rows = 16 * m + lane
                for e0 in range(_EMBED):
                    cols = (e0 + lane) & 63
                    vals = plsc.load_gather(gbufs[j], [rows, cols])
                    res = vals * _SCALE + pb[e0, pl.ds(0, 16)]
                    plsc.store_scatter(
                        sbufs[j],
                        [lax.shift_right_logical(cols, 3), cols & 7, rows],
                        res)

            o_start(u, j)

            @pl.when(u + 2 < 64)
            def _():
                g_start(u + 2, j)

        def upair(i2, carry):
            for j in range(2):
                unit(2 * i2 + j, j)
            return carry

        lax.fori_loop(0, 32, upair, 0)
        o_wait(0)
        o_wait(1)


_k2 = functools.partial(
    pl.kernel,
    mesh=plsc.VectorSubcoreMesh(core_axis_name="c", subcore_axis_name="s"),
    out_type=jax.ShapeDtypeStruct((_WINDOW, 8, _BT, 8, 128), jnp.float32),
    scratch_types=[
        pltpu.VMEM((_WS, _BT, 128), jnp.int32),
        pltpu.VMEM((_WINDOW, _EMBED), jnp.float32),
        pltpu.VMEM((_EMBED, 16), jnp.float32),
        pltpu.VMEM((128, _EMBED), jnp.float32),
        pltpu.VMEM((128, _EMBED), jnp.float32),
        pltpu.VMEM((8, 8, 128), jnp.float32),
        pltpu.VMEM((8, 8, 128), jnp.float32),
        pltpu.SemaphoreType.DMA,
        pltpu.SemaphoreType.DMA,
        pltpu.SemaphoreType.DMA,
        pltpu.SemaphoreType.DMA,
    ],
    compiler_params=pltpu.CompilerParams(
        use_tc_tiling_on_sc=False, needs_layout_passes=False),
)(_k2_body)


def kernel(x, table, pos_enc):
    x4 = jnp.transpose(
        jnp.reshape(jnp.transpose(x.astype(jnp.int32)), (_WT, _WS, _BT, _BL)),
        (0, 2, 1, 3))
    tt = jnp.transpose(table)
    tail = lax.slice(table, (_TAIL0, 0), (_VOC, _EMBED))
    tp = _k1(tt, tail)
    q5 = _k2(tp, x4, pos_enc)
    out = jnp.transpose(
        jnp.reshape(jnp.transpose(q5, (0, 1, 3, 2, 4)),
                    (_WINDOW, _EMBED, _BATCH)),
        (2, 0, 1))
    return out
